# baseline (speedup 1.0000x reference)
import jax
import jax.numpy as jnp
from jax.experimental import pallas as pl
from jax.experimental.pallas import tpu as pltpu

GROUPS = 8
SUB = 8192
DM = 768
TOPK = 32
ENC_BLK = 2048
NB = SUB // ENC_BLK
DEC_BLK = 2048


def _monotonic_key(v):
    b = jax.lax.bitcast_convert_type(v, jnp.int32)
    flip = jax.lax.shift_right_arithmetic(b, 31) & jnp.int32(0x7FFFFFFF)
    return b ^ flip


def _count_ge(key, t):
    return jnp.sum((key >= t).astype(jnp.int32), axis=1, keepdims=True)


def _descend_bits(key, t, start, n):
    def body(i, t):
        cand = t | (jnp.int32(1) << (start - i))
        return jnp.where(_count_ge(key, cand) >= TOPK, cand, t)
    return jax.lax.fori_loop(0, n, body, t)


def _select_acts(key, t, p_ref):
    n_ge = _count_ge(key, t)
    col = jax.lax.broadcasted_iota(jnp.int32, key.shape, 1)
    p_ref[...] = jnp.full(p_ref.shape, jnp.int32(8191))

    @pl.when(jnp.any(n_ge > TOPK))
    def _():
        n_gt = jnp.sum((key > t).astype(jnp.int32), axis=1, keepdims=True)
        m = TOPK - n_gt
        tie = key == t

        def body(i, p):
            cand = p | (jnp.int32(1) << (12 - i))
            cnt = jnp.sum((tie & (col < cand)).astype(jnp.int32), axis=1,
                          keepdims=True)
            return jnp.where(cnt < m, cand, p)

        p = jax.lax.fori_loop(0, 13, body,
                              jnp.zeros((key.shape[0], 1), jnp.int32))
        p_ref[...] = jnp.broadcast_to(p, p_ref.shape)

    sel = (key > t) | ((key == t) & (col <= p_ref[:, 0:1]))
    return jnp.where(sel & (key > 0),
                     jax.lax.bitcast_convert_type(key, jnp.float32), 0.0)


def _encode_body(x_ref, w_ref, b_ref, out_ref, mk_ref, t_ref, p_ref):
    g = pl.program_id(0)
    j = pl.program_id(1)

    @pl.when(g < GROUPS)
    def _():
        pre = jax.lax.dot_general(
            x_ref[...], w_ref[0],
            dimension_numbers=(((1,), (1,)), ((), ())),
            preferred_element_type=jnp.float32,
        ) + b_ref[0]
        par = jax.lax.rem(g, 2)
        mk_ref[par, :, pl.ds(j * ENC_BLK, ENC_BLK)] = _monotonic_key(pre)

    @pl.when(g > 0)
    def _():
        key = mk_ref[jax.lax.rem(g - 1, 2)]
        rows = key.shape[0]

        @pl.when(j == 0)
        def _():
            zero = jnp.zeros((rows, 1), jnp.int32)
            t = jnp.where(_count_ge(key, zero) >= TOPK,
                          zero, jnp.full((rows, 1), jnp.int32(-2147483648)))
            t = _descend_bits(key, t, 30, 7)
            t_ref[...] = jnp.broadcast_to(t, t_ref.shape)

        @pl.when(j == 1)
        def _():
            t = _descend_bits(key, t_ref[:, 0:1], 23, 8)
            t_ref[...] = jnp.broadcast_to(t, t_ref.shape)

        @pl.when(j == 2)
        def _():
            imin = jnp.int32(-2147483648)
            p_hi = t_ref[:, 0:1] | jnp.int32(0xFFFF)
            n_above = jnp.sum((key > p_hi).astype(jnp.int32), axis=1,
                              keepdims=True)
            r = TOPK - n_above
            act = (key >= t_ref[:, 0:1]) & (key <= p_hi)
            done = jnp.zeros((rows, 1), jnp.bool_)
            t_fin = jnp.full((rows, 1), imin)
            for _unused in range(4):
                mval = jnp.max(jnp.where(act, key, imin), axis=1,
                               keepdims=True)
                c = jnp.sum((act & (key == mval)).astype(jnp.int32), axis=1,
                            keepdims=True)
                hit = (~done) & (c >= r)
                t_fin = jnp.where(hit, mval, t_fin)
                done = done | hit
                r = jnp.where(done, r, r - c)
                act = act & (key != mval)
            t_ref[...] = jnp.broadcast_to(t_fin, t_ref.shape)

            @pl.when(jnp.any(~done))
            def _():
                t_fb = _descend_bits(key, p_hi ^ jnp.int32(0xFFFF), 15, 16)
                t_ref[...] = jnp.broadcast_to(
                    jnp.where(done, t_fin, t_fb), t_ref.shape)

        @pl.when(j == 3)
        def _():
            out_ref[...] = _select_acts(key, t_ref[:, 0:1], p_ref)


def _decode_body(a_ref, w_ref, o_ref):
    g = pl.program_id(0)
    j = pl.program_id(1)

    @pl.when((g == 0) & (j == 0))
    def _():
        o_ref[...] = jnp.zeros_like(o_ref)

    o_ref[...] += jax.lax.dot_general(
        a_ref[...], w_ref[0],
        dimension_numbers=(((1,), (1,)), ((), ())),
        preferred_element_type=jnp.float32,
    )


@jax.jit
def kernel(x, enc_W, enc_b, dec_W):
    batch = x.shape[0]
    nb_dec = SUB // DEC_BLK
    enc_b3 = enc_b.reshape(GROUPS, 1, SUB)

    full_acts = pl.pallas_call(
        _encode_body,
        grid=(GROUPS + 1, NB),
        in_specs=[
            pl.BlockSpec((batch, DM), lambda g, j: (0, 0)),
            pl.BlockSpec((1, ENC_BLK, DM),
                         lambda g, j: (jnp.minimum(g, GROUPS - 1),
                                       jnp.where(g < GROUPS, j, NB - 1), 0)),
            pl.BlockSpec((1, 1, ENC_BLK),
                         lambda g, j: (jnp.minimum(g, GROUPS - 1), 0,
                                       jnp.where(g < GROUPS, j, NB - 1))),
        ],
        out_specs=pl.BlockSpec((batch, SUB),
                               lambda g, j: (0, jnp.maximum(g - 1, 0))),
        out_shape=jax.ShapeDtypeStruct((batch, GROUPS * SUB), jnp.float32),
        scratch_shapes=[
            pltpu.VMEM((2, batch, SUB), jnp.int32),
            pltpu.VMEM((batch, 128), jnp.int32),
            pltpu.VMEM((batch, 128), jnp.int32),
        ],
    )(x, enc_W, enc_b3)

    final_recon = pl.pallas_call(
        _decode_body,
        grid=(GROUPS, nb_dec),
        in_specs=[
            pl.BlockSpec((batch, DEC_BLK),
                         lambda g, j: (0, g * (SUB // DEC_BLK) + j)),
            pl.BlockSpec((1, DM, DEC_BLK), lambda g, j: (g, 0, j)),
        ],
        out_specs=pl.BlockSpec((batch, DM), lambda g, j: (0, 0)),
        out_shape=jax.ShapeDtypeStruct((batch, DM), jnp.float32),
    )(full_acts, dec_W)

    return (final_recon, full_acts)


# speedup vs baseline: 1.0431x; 1.0431x over previous
import jax
import jax.numpy as jnp
from jax.experimental import pallas as pl
from jax.experimental.pallas import tpu as pltpu

GROUPS = 8
SUB = 8192
DM = 768
TOPK = 32
ENC_BLK = 2048
NB = SUB // ENC_BLK
DEC_BLK = 2048


def _monotonic_key(v):
    b = jax.lax.bitcast_convert_type(v, jnp.int32)
    flip = jax.lax.shift_right_arithmetic(b, 31) & jnp.int32(0x7FFFFFFF)
    return b ^ flip


def _count_ge(key, t):
    return jnp.sum((key >= t).astype(jnp.int32), axis=1, keepdims=True)


def _descend_bits(key, t, start, n):
    def body(i, t):
        cand = t | (jnp.int32(1) << (start - i))
        return jnp.where(_count_ge(key, cand) >= TOPK, cand, t)
    return jax.lax.fori_loop(0, n, body, t)


def _select_acts(key, t, p_ref):
    n_ge = _count_ge(key, t)
    col = jax.lax.broadcasted_iota(jnp.int32, key.shape, 1)
    p_ref[...] = jnp.full(p_ref.shape, jnp.int32(8191))

    @pl.when(jnp.any(n_ge > TOPK))
    def _():
        n_gt = jnp.sum((key > t).astype(jnp.int32), axis=1, keepdims=True)
        m = TOPK - n_gt
        tie = key == t

        def body(i, p):
            cand = p | (jnp.int32(1) << (12 - i))
            cnt = jnp.sum((tie & (col < cand)).astype(jnp.int32), axis=1,
                          keepdims=True)
            return jnp.where(cnt < m, cand, p)

        p = jax.lax.fori_loop(0, 13, body,
                              jnp.zeros((key.shape[0], 1), jnp.int32))
        p_ref[...] = jnp.broadcast_to(p, p_ref.shape)

    sel = (key > t) | ((key == t) & (col <= p_ref[:, 0:1]))
    return jnp.where(sel & (key > 0),
                     jax.lax.bitcast_convert_type(key, jnp.float32), 0.0)


def _encode_body(x_ref, w_ref, b_ref, out_ref, mk_ref, t_ref, p_ref):
    g = pl.program_id(0)
    j = pl.program_id(1)

    @pl.when(g < GROUPS)
    def _():
        pre = jax.lax.dot_general(
            x_ref[...], w_ref[0],
            dimension_numbers=(((1,), (1,)), ((), ())),
            preferred_element_type=jnp.float32,
        ) + b_ref[0]
        par = jax.lax.rem(g, 2)
        mk_ref[par, :, pl.ds(j * ENC_BLK, ENC_BLK)] = _monotonic_key(pre)

    @pl.when(g > 0)
    def _():
        key = mk_ref[jax.lax.rem(g - 1, 2)]
        rows = key.shape[0]

        @pl.when(j == 0)
        def _():
            zero = jnp.zeros((rows, 1), jnp.int32)
            t = jnp.where(_count_ge(key, zero) >= TOPK,
                          zero, jnp.full((rows, 1), jnp.int32(-2147483648)))
            t = _descend_bits(key, t, 30, 8)
            t_ref[...] = jnp.broadcast_to(t, t_ref.shape)

        @pl.when(j == 1)
        def _():
            t = _descend_bits(key, t_ref[:, 0:1], 22, 8)
            t_ref[...] = jnp.broadcast_to(t, t_ref.shape)

        @pl.when(j == 2)
        def _():
            t = _descend_bits(key, t_ref[:, 0:1], 14, 7)
            imin = jnp.int32(-2147483648)
            p_hi = t | jnp.int32(0xFF)
            n_above = jnp.sum((key > p_hi).astype(jnp.int32), axis=1,
                              keepdims=True)
            r = TOPK - n_above
            act = (key >= t) & (key <= p_hi)
            done = jnp.zeros((rows, 1), jnp.bool_)
            t_fin = jnp.full((rows, 1), imin)
            for _unused in range(3):
                mval = jnp.max(jnp.where(act, key, imin), axis=1,
                               keepdims=True)
                c = jnp.sum((act & (key == mval)).astype(jnp.int32), axis=1,
                            keepdims=True)
                hit = (~done) & (c >= r)
                t_fin = jnp.where(hit, mval, t_fin)
                done = done | hit
                r = jnp.where(done, r, r - c)
                act = act & (key != mval)
            t_ref[...] = jnp.broadcast_to(t_fin, t_ref.shape)

            @pl.when(jnp.any(~done))
            def _():
                t_fb = _descend_bits(key, p_hi ^ jnp.int32(0xFF), 7, 8)
                t_ref[...] = jnp.broadcast_to(
                    jnp.where(done, t_fin, t_fb), t_ref.shape)

        @pl.when(j == 3)
        def _():
            out_ref[...] = _select_acts(key, t_ref[:, 0:1], p_ref)


def _decode_body(a_ref, w_ref, o_ref):
    g = pl.program_id(0)
    j = pl.program_id(1)

    @pl.when((g == 0) & (j == 0))
    def _():
        o_ref[...] = jnp.zeros_like(o_ref)

    o_ref[...] += jax.lax.dot_general(
        a_ref[...], w_ref[0],
        dimension_numbers=(((1,), (1,)), ((), ())),
        preferred_element_type=jnp.float32,
    )


@jax.jit
def kernel(x, enc_W, enc_b, dec_W):
    batch = x.shape[0]
    nb_dec = SUB // DEC_BLK
    enc_b3 = enc_b.reshape(GROUPS, 1, SUB)

    full_acts = pl.pallas_call(
        _encode_body,
        grid=(GROUPS + 1, NB),
        in_specs=[
            pl.BlockSpec((batch, DM), lambda g, j: (0, 0)),
            pl.BlockSpec((1, ENC_BLK, DM),
                         lambda g, j: (jnp.minimum(g, GROUPS - 1),
                                       jnp.where(g < GROUPS, j, NB - 1), 0)),
            pl.BlockSpec((1, 1, ENC_BLK),
                         lambda g, j: (jnp.minimum(g, GROUPS - 1), 0,
                                       jnp.where(g < GROUPS, j, NB - 1))),
        ],
        out_specs=pl.BlockSpec((batch, SUB),
                               lambda g, j: (0, jnp.maximum(g - 1, 0))),
        out_shape=jax.ShapeDtypeStruct((batch, GROUPS * SUB), jnp.float32),
        scratch_shapes=[
            pltpu.VMEM((2, batch, SUB), jnp.int32),
            pltpu.VMEM((batch, 128), jnp.int32),
            pltpu.VMEM((batch, 128), jnp.int32),
        ],
    )(x, enc_W, enc_b3)

    final_recon = pl.pallas_call(
        _decode_body,
        grid=(GROUPS, nb_dec),
        in_specs=[
            pl.BlockSpec((batch, DEC_BLK),
                         lambda g, j: (0, g * (SUB // DEC_BLK) + j)),
            pl.BlockSpec((1, DM, DEC_BLK), lambda g, j: (g, 0, j)),
        ],
        out_specs=pl.BlockSpec((batch, DM), lambda g, j: (0, 0)),
        out_shape=jax.ShapeDtypeStruct((batch, DM), jnp.float32),
    )(full_acts, dec_W)

    return (final_recon, full_acts)


# speedup vs baseline: 1.1716x; 1.1232x over previous
import jax
import jax.numpy as jnp
from jax.experimental import pallas as pl
from jax.experimental.pallas import tpu as pltpu

GROUPS = 8
SUB = 8192
DM = 768
TOPK = 32
ENC_BLK = 2048
NB = SUB // ENC_BLK
DEC_BLK = 2048


def _monotonic_key(v):
    b = jax.lax.bitcast_convert_type(v, jnp.int32)
    flip = jax.lax.shift_right_arithmetic(b, 31) & jnp.int32(0x7FFFFFFF)
    return b ^ flip


def _count_ge(key, t):
    return jnp.sum((key >= t).astype(jnp.int32), axis=1, keepdims=True)


def _descend_bits(key, t, start, n):
    def body(i, t):
        cand = t | (jnp.int32(1) << (start - i))
        return jnp.where(_count_ge(key, cand) >= TOPK, cand, t)
    return jax.lax.fori_loop(0, n, body, t)


def _select_acts(key, t, p_ref):
    n_ge = _count_ge(key, t)
    col = jax.lax.broadcasted_iota(jnp.int32, key.shape, 1)
    p_ref[...] = jnp.full(p_ref.shape, jnp.int32(8191))

    @pl.when(jnp.any(n_ge > TOPK))
    def _():
        n_gt = jnp.sum((key > t).astype(jnp.int32), axis=1, keepdims=True)
        m = TOPK - n_gt
        tie = key == t

        def body(i, p):
            cand = p | (jnp.int32(1) << (12 - i))
            cnt = jnp.sum((tie & (col < cand)).astype(jnp.int32), axis=1,
                          keepdims=True)
            return jnp.where(cnt < m, cand, p)

        p = jax.lax.fori_loop(0, 13, body,
                              jnp.zeros((key.shape[0], 1), jnp.int32))
        p_ref[...] = jnp.broadcast_to(p, p_ref.shape)

    sel = (key > t) | ((key == t) & (col <= p_ref[:, 0:1]))
    return jnp.where(sel & (key > 0),
                     jax.lax.bitcast_convert_type(key, jnp.float32), 0.0)


def _fused_body(x_ref, w_ref, b_ref, d_ref, out_ref, recon_ref,
                mk_ref, t_ref, p_ref, av_ref):
    g = pl.program_id(0)
    j = pl.program_id(1)

    @pl.when(g < GROUPS)
    def _():
        pre = jax.lax.dot_general(
            x_ref[...], w_ref[0],
            dimension_numbers=(((1,), (1,)), ((), ())),
            preferred_element_type=jnp.float32,
        ) + b_ref[0]
        par = jax.lax.rem(g, 2)
        mk_ref[par, :, pl.ds(j * ENC_BLK, ENC_BLK)] = _monotonic_key(pre)

    @pl.when((g > 0) & (g <= GROUPS))
    def _():
        key = mk_ref[jax.lax.rem(g - 1, 2)]
        rows = key.shape[0]

        @pl.when(j == 0)
        def _():
            zero = jnp.zeros((rows, 1), jnp.int32)
            t = jnp.where(_count_ge(key, zero) >= TOPK,
                          zero, jnp.full((rows, 1), jnp.int32(-2147483648)))
            t = _descend_bits(key, t, 30, 8)
            t_ref[...] = jnp.broadcast_to(t, t_ref.shape)

        @pl.when(j == 1)
        def _():
            t = _descend_bits(key, t_ref[:, 0:1], 22, 8)
            t_ref[...] = jnp.broadcast_to(t, t_ref.shape)

        @pl.when(j == 2)
        def _():
            t = _descend_bits(key, t_ref[:, 0:1], 14, 7)
            imin = jnp.int32(-2147483648)
            p_hi = t | jnp.int32(0xFF)
            n_above = jnp.sum((key > p_hi).astype(jnp.int32), axis=1,
                              keepdims=True)
            r = TOPK - n_above
            act = (key >= t) & (key <= p_hi)
            done = jnp.zeros((rows, 1), jnp.bool_)
            t_fin = jnp.full((rows, 1), imin)
            for _unused in range(3):
                mval = jnp.max(jnp.where(act, key, imin), axis=1,
                               keepdims=True)
                c = jnp.sum((act & (key == mval)).astype(jnp.int32), axis=1,
                            keepdims=True)
                hit = (~done) & (c >= r)
                t_fin = jnp.where(hit, mval, t_fin)
                done = done | hit
                r = jnp.where(done, r, r - c)
                act = act & (key != mval)
            t_ref[...] = jnp.broadcast_to(t_fin, t_ref.shape)

            @pl.when(jnp.any(~done))
            def _():
                t_fb = _descend_bits(key, p_hi ^ jnp.int32(0xFF), 7, 8)
                t_ref[...] = jnp.broadcast_to(
                    jnp.where(done, t_fin, t_fb), t_ref.shape)

        @pl.when(j == 3)
        def _():
            acts = _select_acts(key, t_ref[:, 0:1], p_ref)
            out_ref[...] = acts
            av_ref[jax.lax.rem(g - 1, 2)] = acts

    @pl.when(g >= 2)
    def _():
        @pl.when((g == 2) & (j == 0))
        def _():
            recon_ref[...] = jnp.zeros_like(recon_ref)

        a_blk = av_ref[jax.lax.rem(g - 2, 2), :, pl.ds(j * ENC_BLK, ENC_BLK)]
        recon_ref[...] += jax.lax.dot_general(
            a_blk, d_ref[0],
            dimension_numbers=(((1,), (1,)), ((), ())),
            preferred_element_type=jnp.float32,
        )


@jax.jit
def kernel(x, enc_W, enc_b, dec_W):
    batch = x.shape[0]
    enc_b3 = enc_b.reshape(GROUPS, 1, SUB)

    full_acts, final_recon = pl.pallas_call(
        _fused_body,
        grid=(GROUPS + 2, NB),
        in_specs=[
            pl.BlockSpec((batch, DM), lambda g, j: (0, 0)),
            pl.BlockSpec((1, ENC_BLK, DM),
                         lambda g, j: (jnp.minimum(g, GROUPS - 1),
                                       jnp.where(g < GROUPS, j, NB - 1), 0)),
            pl.BlockSpec((1, 1, ENC_BLK),
                         lambda g, j: (jnp.minimum(g, GROUPS - 1), 0,
                                       jnp.where(g < GROUPS, j, NB - 1))),
            pl.BlockSpec((1, DM, ENC_BLK),
                         lambda g, j: (jnp.maximum(g - 2, 0), 0,
                                       jnp.where(g >= 2, j, 0))),
        ],
        out_specs=[
            pl.BlockSpec((batch, SUB),
                         lambda g, j: (0, jnp.clip(g - 1, 0, GROUPS - 1))),
            pl.BlockSpec((batch, DM), lambda g, j: (0, 0)),
        ],
        out_shape=[
            jax.ShapeDtypeStruct((batch, GROUPS * SUB), jnp.float32),
            jax.ShapeDtypeStruct((batch, DM), jnp.float32),
        ],
        scratch_shapes=[
            pltpu.VMEM((2, batch, SUB), jnp.int32),
            pltpu.VMEM((batch, 128), jnp.int32),
            pltpu.VMEM((batch, 128), jnp.int32),
            pltpu.VMEM((2, batch, SUB), jnp.float32),
        ],
    )(x, enc_W, enc_b3, dec_W)

    return (final_recon, full_acts)


# speedup vs baseline: 1.1919x; 1.0173x over previous
import jax
import jax.numpy as jnp
from jax.experimental import pallas as pl
from jax.experimental.pallas import tpu as pltpu

GROUPS = 8
SUB = 8192
DM = 768
TOPK = 32
ENC_BLK = 2048
NB = SUB // ENC_BLK
DEC_BLK = 2048


def _monotonic_key(v):
    b = jax.lax.bitcast_convert_type(v, jnp.int32)
    flip = jax.lax.shift_right_arithmetic(b, 31) & jnp.int32(0x7FFFFFFF)
    return b ^ flip


def _count_ge(key, t):
    return jnp.sum((key >= t).astype(jnp.int32), axis=1, keepdims=True)


def _descend_bits(key, t, start, n):
    def body(i, t):
        cand = t | (jnp.int32(1) << (start - i))
        return jnp.where(_count_ge(key, cand) >= TOPK, cand, t)
    return jax.lax.fori_loop(0, n, body, t)


def _count_ge_bf16(bkey, cand):
    c = jax.lax.bitcast_convert_type(cand, jnp.float32).astype(jnp.bfloat16)
    a = jnp.where(bkey >= c, jnp.bfloat16(1), jnp.bfloat16(0))
    w = a.shape[1]
    while w > 128:
        a = a[:, : w // 2] + a[:, w // 2:]
        w //= 2
    return jnp.sum(a.astype(jnp.float32), axis=1, keepdims=True)


def _descend_bits_bf16(bkey, t, start, n):
    def body(i, t):
        cand = t | (jnp.int32(1) << (start - i))
        return jnp.where(_count_ge_bf16(bkey, cand) >= TOPK, cand, t)
    return jax.lax.fori_loop(0, n, body, t)


def _select_acts(key, t, p_ref):
    n_ge = _count_ge(key, t)
    col = jax.lax.broadcasted_iota(jnp.int32, key.shape, 1)
    p_ref[...] = jnp.full(p_ref.shape, jnp.int32(8191))

    @pl.when(jnp.any(n_ge > TOPK))
    def _():
        n_gt = jnp.sum((key > t).astype(jnp.int32), axis=1, keepdims=True)
        m = TOPK - n_gt
        tie = key == t

        def body(i, p):
            cand = p | (jnp.int32(1) << (12 - i))
            cnt = jnp.sum((tie & (col < cand)).astype(jnp.int32), axis=1,
                          keepdims=True)
            return jnp.where(cnt < m, cand, p)

        p = jax.lax.fori_loop(0, 13, body,
                              jnp.zeros((key.shape[0], 1), jnp.int32))
        p_ref[...] = jnp.broadcast_to(p, p_ref.shape)

    sel = (key > t) | ((key == t) & (col <= p_ref[:, 0:1]))
    return jnp.where(sel & (key > 0),
                     jax.lax.bitcast_convert_type(key, jnp.float32), 0.0)


def _fused_body(x_ref, w_ref, b_ref, d_ref, out_ref, recon_ref,
                mk_ref, t_ref, p_ref, av_ref, bk_ref):
    g = pl.program_id(0)
    j = pl.program_id(1)

    @pl.when(g < GROUPS)
    def _():
        pre = jax.lax.dot_general(
            x_ref[...], w_ref[0],
            dimension_numbers=(((1,), (1,)), ((), ())),
            preferred_element_type=jnp.float32,
        ) + b_ref[0]
        par = jax.lax.rem(g, 2)
        mk_ref[par, :, pl.ds(j * ENC_BLK, ENC_BLK)] = _monotonic_key(pre)
        bits = jax.lax.bitcast_convert_type(pre, jnp.int32)
        trunc = jax.lax.bitcast_convert_type(
            bits & jnp.int32(-65536), jnp.float32)
        bk_ref[par, :, pl.ds(j * ENC_BLK, ENC_BLK)] = trunc.astype(
            jnp.bfloat16)

    @pl.when((g > 0) & (g <= GROUPS))
    def _():
        par = jax.lax.rem(g - 1, 2)
        key = mk_ref[par]
        bkey = bk_ref[par]
        rows = key.shape[0]

        @pl.when(j == 0)
        def _():
            zero = jnp.zeros((rows, 1), jnp.int32)
            t = jnp.where(_count_ge(key, zero) >= TOPK,
                          zero, jnp.full((rows, 1), jnp.int32(-2147483648)))
            t_ref[...] = jnp.broadcast_to(t, t_ref.shape)

            @pl.when(jnp.all(t >= 0))
            def _():
                tb = _descend_bits_bf16(bkey, t, 30, 7)
                t_ref[...] = jnp.broadcast_to(tb, t_ref.shape)

            @pl.when(jnp.any(t < 0))
            def _():
                tf = _descend_bits(key, t, 30, 7)
                t_ref[...] = jnp.broadcast_to(tf, t_ref.shape)

        @pl.when(j == 1)
        def _():
            t = t_ref[:, 0:1]

            @pl.when(jnp.all(t >= (jnp.int32(1) << 24)))
            def _():
                tb = _descend_bits_bf16(bkey, t, 23, 8)
                t_ref[...] = jnp.broadcast_to(tb, t_ref.shape)

            @pl.when(jnp.any(t < (jnp.int32(1) << 24)))
            def _():
                tf = _descend_bits(key, t, 23, 8)
                t_ref[...] = jnp.broadcast_to(tf, t_ref.shape)

        @pl.when(j == 2)
        def _():
            t = _descend_bits(key, t_ref[:, 0:1], 15, 8)
            imin = jnp.int32(-2147483648)
            p_hi = t | jnp.int32(0xFF)
            n_above = jnp.sum((key > p_hi).astype(jnp.int32), axis=1,
                              keepdims=True)
            r = TOPK - n_above
            act = (key >= t) & (key <= p_hi)
            done = jnp.zeros((rows, 1), jnp.bool_)
            t_fin = jnp.full((rows, 1), imin)
            for _unused in range(3):
                mval = jnp.max(jnp.where(act, key, imin), axis=1,
                               keepdims=True)
                c = jnp.sum((act & (key == mval)).astype(jnp.int32), axis=1,
                            keepdims=True)
                hit = (~done) & (c >= r)
                t_fin = jnp.where(hit, mval, t_fin)
                done = done | hit
                r = jnp.where(done, r, r - c)
                act = act & (key != mval)
            t_ref[...] = jnp.broadcast_to(t_fin, t_ref.shape)

            @pl.when(jnp.any(~done))
            def _():
                t_fb = _descend_bits(key, p_hi ^ jnp.int32(0xFF), 7, 8)
                t_ref[...] = jnp.broadcast_to(
                    jnp.where(done, t_fin, t_fb), t_ref.shape)

        @pl.when(j == 3)
        def _():
            acts = _select_acts(key, t_ref[:, 0:1], p_ref)
            out_ref[...] = acts
            av_ref[jax.lax.rem(g - 1, 2)] = acts

    @pl.when(g >= 2)
    def _():
        @pl.when((g == 2) & (j == 0))
        def _():
            recon_ref[...] = jnp.zeros_like(recon_ref)

        a_blk = av_ref[jax.lax.rem(g - 2, 2), :, pl.ds(j * ENC_BLK, ENC_BLK)]
        recon_ref[...] += jax.lax.dot_general(
            a_blk, d_ref[0],
            dimension_numbers=(((1,), (1,)), ((), ())),
            preferred_element_type=jnp.float32,
        )


@jax.jit
def kernel(x, enc_W, enc_b, dec_W):
    batch = x.shape[0]
    enc_b3 = enc_b.reshape(GROUPS, 1, SUB)

    full_acts, final_recon = pl.pallas_call(
        _fused_body,
        grid=(GROUPS + 2, NB),
        in_specs=[
            pl.BlockSpec((batch, DM), lambda g, j: (0, 0)),
            pl.BlockSpec((1, ENC_BLK, DM),
                         lambda g, j: (jnp.minimum(g, GROUPS - 1),
                                       jnp.where(g < GROUPS, j, NB - 1), 0)),
            pl.BlockSpec((1, 1, ENC_BLK),
                         lambda g, j: (jnp.minimum(g, GROUPS - 1), 0,
                                       jnp.where(g < GROUPS, j, NB - 1))),
            pl.BlockSpec((1, DM, ENC_BLK),
                         lambda g, j: (jnp.maximum(g - 2, 0), 0,
                                       jnp.where(g >= 2, j, 0))),
        ],
        out_specs=[
            pl.BlockSpec((batch, SUB),
                         lambda g, j: (0, jnp.clip(g - 1, 0, GROUPS - 1))),
            pl.BlockSpec((batch, DM), lambda g, j: (0, 0)),
        ],
        out_shape=[
            jax.ShapeDtypeStruct((batch, GROUPS * SUB), jnp.float32),
            jax.ShapeDtypeStruct((batch, DM), jnp.float32),
        ],
        scratch_shapes=[
            pltpu.VMEM((2, batch, SUB), jnp.int32),
            pltpu.VMEM((batch, 128), jnp.int32),
            pltpu.VMEM((batch, 128), jnp.int32),
            pltpu.VMEM((2, batch, SUB), jnp.float32),
            pltpu.VMEM((2, batch, SUB), jnp.bfloat16),
        ],
    )(x, enc_W, enc_b3, dec_W)

    return (final_recon, full_acts)


# speedup vs baseline: 1.2212x; 1.0246x over previous
import jax
import jax.numpy as jnp
from jax.experimental import pallas as pl
from jax.experimental.pallas import tpu as pltpu

GROUPS = 8
SUB = 8192
DM = 768
TOPK = 32
ENC_BLK = 2048
NB = SUB // ENC_BLK
DEC_BLK = 2048


def _monotonic_key(v):
    b = jax.lax.bitcast_convert_type(v, jnp.int32)
    flip = jax.lax.shift_right_arithmetic(b, 31) & jnp.int32(0x7FFFFFFF)
    return b ^ flip


def _count_ge(key, t):
    return jnp.sum((key >= t).astype(jnp.int32), axis=1, keepdims=True)


def _descend_bits(key, t, start, n):
    def body(i, t):
        cand = t | (jnp.int32(1) << (start - i))
        return jnp.where(_count_ge(key, cand) >= TOPK, cand, t)
    return jax.lax.fori_loop(0, n, body, t)


def _count_ge_bf16(bkey, cand):
    c = jax.lax.bitcast_convert_type(cand, jnp.float32).astype(jnp.bfloat16)
    a = jnp.where(bkey >= c, jnp.bfloat16(1), jnp.bfloat16(0))
    w = a.shape[1]
    while w > 128:
        a = a[:, : w // 2] + a[:, w // 2:]
        w //= 2
    return jnp.sum(a.astype(jnp.float32), axis=1, keepdims=True)


def _descend_bits_bf16(bkey, t, start, n):
    def body(i, t):
        cand = t | (jnp.int32(1) << (start - i))
        return jnp.where(_count_ge_bf16(bkey, cand) >= TOPK, cand, t)
    return jax.lax.fori_loop(0, n, body, t)


def _select_acts(key, t, p_ref):
    n_ge = _count_ge(key, t)
    col = jax.lax.broadcasted_iota(jnp.int32, key.shape, 1)
    p_ref[...] = jnp.full(p_ref.shape, jnp.int32(8191))

    @pl.when(jnp.any(n_ge > TOPK))
    def _():
        n_gt = jnp.sum((key > t).astype(jnp.int32), axis=1, keepdims=True)
        m = TOPK - n_gt
        tie = key == t

        def body(i, p):
            cand = p | (jnp.int32(1) << (12 - i))
            cnt = jnp.sum((tie & (col < cand)).astype(jnp.int32), axis=1,
                          keepdims=True)
            return jnp.where(cnt < m, cand, p)

        p = jax.lax.fori_loop(0, 13, body,
                              jnp.zeros((key.shape[0], 1), jnp.int32))
        p_ref[...] = jnp.broadcast_to(p, p_ref.shape)

    sel = (key > t) | ((key == t) & (col <= p_ref[:, 0:1]))
    return jnp.where(sel & (key > 0),
                     jax.lax.bitcast_convert_type(key, jnp.float32), 0.0)


def _fused_body(x_ref, w_ref, b_ref, d_ref, out_ref, recon_ref,
                mk_ref, t_ref, p_ref, av_ref, bk_ref):
    g = pl.program_id(0)
    j = pl.program_id(1)

    @pl.when(g < GROUPS)
    def _():
        pre = jax.lax.dot_general(
            x_ref[...], w_ref[0],
            dimension_numbers=(((1,), (1,)), ((), ())),
            preferred_element_type=jnp.float32,
        ) + b_ref[0]
        par = jax.lax.rem(g, 2)
        mk_ref[par, :, pl.ds(j * ENC_BLK, ENC_BLK)] = _monotonic_key(pre)
        bits = jax.lax.bitcast_convert_type(pre, jnp.int32)
        trunc = jax.lax.bitcast_convert_type(
            bits & jnp.int32(-65536), jnp.float32)
        bk_ref[par, :, pl.ds(j * ENC_BLK, ENC_BLK)] = trunc.astype(
            jnp.bfloat16)

    @pl.when((g > 0) & (g <= GROUPS))
    def _():
        par = jax.lax.rem(g - 1, 2)
        key = mk_ref[par]
        bkey = bk_ref[par]
        rows = key.shape[0]

        @pl.when(j == 0)
        def _():
            zero = jnp.zeros((rows, 1), jnp.int32)
            t = jnp.where(_count_ge(key, zero) >= TOPK,
                          zero, jnp.full((rows, 1), jnp.int32(-2147483648)))
            t_ref[...] = jnp.broadcast_to(t, t_ref.shape)

            @pl.when(jnp.all(t >= 0))
            def _():
                tb = _descend_bits_bf16(bkey, t, 30, 7)
                t_ref[...] = jnp.broadcast_to(tb, t_ref.shape)

            @pl.when(jnp.any(t < 0))
            def _():
                tf = _descend_bits(key, t, 30, 7)
                t_ref[...] = jnp.broadcast_to(tf, t_ref.shape)

        @pl.when(j == 1)
        def _():
            t = t_ref[:, 0:1]

            @pl.when(jnp.all(t >= (jnp.int32(1) << 24)))
            def _():
                tb = _descend_bits_bf16(bkey, t, 23, 8)
                t_ref[...] = jnp.broadcast_to(tb, t_ref.shape)

            @pl.when(jnp.any(t < (jnp.int32(1) << 24)))
            def _():
                tf = _descend_bits(key, t, 23, 8)
                t_ref[...] = jnp.broadcast_to(tf, t_ref.shape)

        @pl.when(j == 2)
        def _():
            t = _descend_bits(key, t_ref[:, 0:1], 15, 8)
            t_ref[...] = jnp.broadcast_to(t, t_ref.shape)

        @pl.when(j == 3)
        def _():
            t = t_ref[:, 0:1]
            imin = jnp.int32(-2147483648)
            p_hi = t | jnp.int32(0xFF)
            n_above = jnp.sum((key > p_hi).astype(jnp.int32), axis=1,
                              keepdims=True)
            r = TOPK - n_above
            act = (key >= t) & (key <= p_hi)
            done = jnp.zeros((rows, 1), jnp.bool_)
            t_fin = jnp.full((rows, 1), imin)
            for _unused in range(3):
                mval = jnp.max(jnp.where(act, key, imin), axis=1,
                               keepdims=True)
                c = jnp.sum((act & (key == mval)).astype(jnp.int32), axis=1,
                            keepdims=True)
                hit = (~done) & (c >= r)
                t_fin = jnp.where(hit, mval, t_fin)
                done = done | hit
                r = jnp.where(done, r, r - c)
                act = act & (key != mval)
            t_ref[...] = jnp.broadcast_to(t_fin, t_ref.shape)

            @pl.when(jnp.any(~done))
            def _():
                t_fb = _descend_bits(key, p_hi ^ jnp.int32(0xFF), 7, 8)
                t_ref[...] = jnp.broadcast_to(
                    jnp.where(done, t_fin, t_fb), t_ref.shape)

        @pl.when(j == 3)
        def _():
            acts = _select_acts(key, t_ref[:, 0:1], p_ref)
            out_ref[...] = acts
            av_ref[jax.lax.rem(g - 1, 2)] = acts

    @pl.when(g >= 2)
    def _():
        @pl.when((g == 2) & (j == 0))
        def _():
            recon_ref[...] = jnp.zeros_like(recon_ref)

        a_blk = av_ref[jax.lax.rem(g - 2, 2), :, pl.ds(j * ENC_BLK, ENC_BLK)]
        recon_ref[...] += jax.lax.dot_general(
            a_blk, d_ref[0],
            dimension_numbers=(((1,), (1,)), ((), ())),
            preferred_element_type=jnp.float32,
        )


@jax.jit
def kernel(x, enc_W, enc_b, dec_W):
    batch = x.shape[0]
    enc_b3 = enc_b.reshape(GROUPS, 1, SUB)

    full_acts, final_recon = pl.pallas_call(
        _fused_body,
        grid=(GROUPS + 2, NB),
        in_specs=[
            pl.BlockSpec((batch, DM), lambda g, j: (0, 0)),
            pl.BlockSpec((1, ENC_BLK, DM),
                         lambda g, j: (jnp.minimum(g, GROUPS - 1),
                                       jnp.where(g < GROUPS, j, NB - 1), 0)),
            pl.BlockSpec((1, 1, ENC_BLK),
                         lambda g, j: (jnp.minimum(g, GROUPS - 1), 0,
                                       jnp.where(g < GROUPS, j, NB - 1))),
            pl.BlockSpec((1, DM, ENC_BLK),
                         lambda g, j: (jnp.maximum(g - 2, 0), 0,
                                       jnp.where(g >= 2, j, 0))),
        ],
        out_specs=[
            pl.BlockSpec((batch, SUB),
                         lambda g, j: (0, jnp.clip(g - 1, 0, GROUPS - 1))),
            pl.BlockSpec((batch, DM), lambda g, j: (0, 0)),
        ],
        out_shape=[
            jax.ShapeDtypeStruct((batch, GROUPS * SUB), jnp.float32),
            jax.ShapeDtypeStruct((batch, DM), jnp.float32),
        ],
        scratch_shapes=[
            pltpu.VMEM((2, batch, SUB), jnp.int32),
            pltpu.VMEM((batch, 128), jnp.int32),
            pltpu.VMEM((batch, 128), jnp.int32),
            pltpu.VMEM((2, batch, SUB), jnp.float32),
            pltpu.VMEM((2, batch, SUB), jnp.bfloat16),
        ],
    )(x, enc_W, enc_b3, dec_W)

    return (final_recon, full_acts)


# speedup vs baseline: 1.2255x; 1.0035x over previous
import jax
import jax.numpy as jnp
from jax.experimental import pallas as pl
from jax.experimental.pallas import tpu as pltpu

GROUPS = 8
SUB = 8192
DM = 768
TOPK = 32
ENC_BLK = 2048
NB = SUB // ENC_BLK
DEC_BLK = 2048


def _monotonic_key(v):
    b = jax.lax.bitcast_convert_type(v, jnp.int32)
    flip = jax.lax.shift_right_arithmetic(b, 31) & jnp.int32(0x7FFFFFFF)
    return b ^ flip


def _count_ge(key, t):
    return jnp.sum((key >= t).astype(jnp.int32), axis=1, keepdims=True)


def _descend_bits(key, t, start, n):
    def body(i, t):
        cand = t | (jnp.int32(1) << (start - i))
        return jnp.where(_count_ge(key, cand) >= TOPK, cand, t)
    return jax.lax.fori_loop(0, n, body, t)


def _count_ge_bf16(bkey, cand):
    c = jax.lax.bitcast_convert_type(cand, jnp.float32).astype(jnp.bfloat16)
    a = jnp.where(bkey >= c, jnp.bfloat16(1), jnp.bfloat16(0))
    w = a.shape[1]
    while w > 128:
        a = a[:, : w // 2] + a[:, w // 2:]
        w //= 2
    return jnp.sum(a.astype(jnp.float32), axis=1, keepdims=True)


def _descend_bits_bf16(bkey, t, start, n):
    def body(i, t):
        cand = t | (jnp.int32(1) << (start - i))
        return jnp.where(_count_ge_bf16(bkey, cand) >= TOPK, cand, t)
    return jax.lax.fori_loop(0, n, body, t)


def _select_acts(key, t, p_ref):
    n_ge = _count_ge(key, t)
    col = jax.lax.broadcasted_iota(jnp.int32, key.shape, 1)
    p_ref[...] = jnp.full(p_ref.shape, jnp.int32(8191))

    @pl.when(jnp.any(n_ge > TOPK))
    def _():
        n_gt = jnp.sum((key > t).astype(jnp.int32), axis=1, keepdims=True)
        m = TOPK - n_gt
        tie = key == t

        def body(i, p):
            cand = p | (jnp.int32(1) << (12 - i))
            cnt = jnp.sum((tie & (col < cand)).astype(jnp.int32), axis=1,
                          keepdims=True)
            return jnp.where(cnt < m, cand, p)

        p = jax.lax.fori_loop(0, 13, body,
                              jnp.zeros((key.shape[0], 1), jnp.int32))
        p_ref[...] = jnp.broadcast_to(p, p_ref.shape)

    sel = (key > t) | ((key == t) & (col <= p_ref[:, 0:1]))
    return jnp.where(sel & (key > 0),
                     jax.lax.bitcast_convert_type(key, jnp.float32), 0.0)


def _fused_body(x_ref, w_ref, b_ref, d_ref, out_ref, recon_ref,
                mk_ref, t_ref, p_ref, av_ref, bk_ref):
    g = pl.program_id(0)
    j = pl.program_id(1)

    @pl.when(g < GROUPS)
    def _():
        pre = jax.lax.dot_general(
            x_ref[...], w_ref[0],
            dimension_numbers=(((1,), (1,)), ((), ())),
            preferred_element_type=jnp.float32,
        ) + b_ref[0]
        par = jax.lax.rem(g, 2)
        mk_ref[par, :, pl.ds(j * ENC_BLK, ENC_BLK)] = _monotonic_key(pre)
        bits = jax.lax.bitcast_convert_type(pre, jnp.int32)
        trunc = jax.lax.bitcast_convert_type(
            bits & jnp.int32(-65536), jnp.float32)
        bk_ref[par, :, pl.ds(j * ENC_BLK, ENC_BLK)] = trunc.astype(
            jnp.bfloat16)

    @pl.when((g > 0) & (g <= GROUPS))
    def _():
        par = jax.lax.rem(g - 1, 2)
        key = mk_ref[par]
        bkey = bk_ref[par]
        rows = key.shape[0]

        @pl.when(j == 0)
        def _():
            zero = jnp.zeros((rows, 1), jnp.int32)
            t = jnp.where(_count_ge(key, zero) >= TOPK,
                          zero, jnp.full((rows, 1), jnp.int32(-2147483648)))
            t_ref[...] = jnp.broadcast_to(t, t_ref.shape)

            @pl.when(jnp.all(t >= 0))
            def _():
                tb = _descend_bits_bf16(bkey, t, 30, 7)
                t_ref[...] = jnp.broadcast_to(tb, t_ref.shape)

            @pl.when(jnp.any(t < 0))
            def _():
                tf = _descend_bits(key, t, 30, 7)
                t_ref[...] = jnp.broadcast_to(tf, t_ref.shape)

        @pl.when(j == 1)
        def _():
            t = t_ref[:, 0:1]

            @pl.when(jnp.all(t >= (jnp.int32(1) << 24)))
            def _():
                tb = _descend_bits_bf16(bkey, t, 23, 8)
                t_ref[...] = jnp.broadcast_to(tb, t_ref.shape)

            @pl.when(jnp.any(t < (jnp.int32(1) << 24)))
            def _():
                tf = _descend_bits(key, t, 23, 8)
                t_ref[...] = jnp.broadcast_to(tf, t_ref.shape)

        @pl.when(j == 2)
        def _():
            t = _descend_bits(key, t_ref[:, 0:1], 15, 4)
            t_ref[...] = jnp.broadcast_to(t, t_ref.shape)

        @pl.when(j == 3)
        def _():
            t = t_ref[:, 0:1]
            imin = jnp.int32(-2147483648)
            p_hi = t | jnp.int32(0xFFF)
            n_above = jnp.sum((key > p_hi).astype(jnp.int32), axis=1,
                              keepdims=True)
            r = TOPK - n_above
            act = (key >= t) & (key <= p_hi)
            done = jnp.zeros((rows, 1), jnp.bool_)
            t_fin = jnp.full((rows, 1), imin)
            for _unused in range(3):
                mval = jnp.max(jnp.where(act, key, imin), axis=1,
                               keepdims=True)
                c = jnp.sum((act & (key == mval)).astype(jnp.int32), axis=1,
                            keepdims=True)
                hit = (~done) & (c >= r)
                t_fin = jnp.where(hit, mval, t_fin)
                done = done | hit
                r = jnp.where(done, r, r - c)
                act = act & (key != mval)
            t_ref[...] = jnp.broadcast_to(t_fin, t_ref.shape)

            @pl.when(jnp.any(~done))
            def _():
                t_fb = _descend_bits(key, p_hi ^ jnp.int32(0xFFF), 11, 12)
                t_ref[...] = jnp.broadcast_to(
                    jnp.where(done, t_fin, t_fb), t_ref.shape)

        @pl.when(j == 3)
        def _():
            acts = _select_acts(key, t_ref[:, 0:1], p_ref)
            out_ref[...] = acts
            av_ref[jax.lax.rem(g - 1, 2)] = acts

    @pl.when(g >= 2)
    def _():
        @pl.when((g == 2) & (j == 0))
        def _():
            recon_ref[...] = jnp.zeros_like(recon_ref)

        a_blk = av_ref[jax.lax.rem(g - 2, 2), :, pl.ds(j * ENC_BLK, ENC_BLK)]
        recon_ref[...] += jax.lax.dot_general(
            a_blk, d_ref[0],
            dimension_numbers=(((1,), (1,)), ((), ())),
            preferred_element_type=jnp.float32,
        )


@jax.jit
def kernel(x, enc_W, enc_b, dec_W):
    batch = x.shape[0]
    enc_b3 = enc_b.reshape(GROUPS, 1, SUB)

    full_acts, final_recon = pl.pallas_call(
        _fused_body,
        grid=(GROUPS + 2, NB),
        in_specs=[
            pl.BlockSpec((batch, DM), lambda g, j: (0, 0)),
            pl.BlockSpec((1, ENC_BLK, DM),
                         lambda g, j: (jnp.minimum(g, GROUPS - 1),
                                       jnp.where(g < GROUPS, j, NB - 1), 0)),
            pl.BlockSpec((1, 1, ENC_BLK),
                         lambda g, j: (jnp.minimum(g, GROUPS - 1), 0,
                                       jnp.where(g < GROUPS, j, NB - 1))),
            pl.BlockSpec((1, DM, ENC_BLK),
                         lambda g, j: (jnp.maximum(g - 2, 0), 0,
                                       jnp.where(g >= 2, j, 0))),
        ],
        out_specs=[
            pl.BlockSpec((batch, SUB),
                         lambda g, j: (0, jnp.clip(g - 1, 0, GROUPS - 1))),
            pl.BlockSpec((batch, DM), lambda g, j: (0, 0)),
        ],
        out_shape=[
            jax.ShapeDtypeStruct((batch, GROUPS * SUB), jnp.float32),
            jax.ShapeDtypeStruct((batch, DM), jnp.float32),
        ],
        scratch_shapes=[
            pltpu.VMEM((2, batch, SUB), jnp.int32),
            pltpu.VMEM((batch, 128), jnp.int32),
            pltpu.VMEM((batch, 128), jnp.int32),
            pltpu.VMEM((2, batch, SUB), jnp.float32),
            pltpu.VMEM((2, batch, SUB), jnp.bfloat16),
        ],
    )(x, enc_W, enc_b3, dec_W)

    return (final_recon, full_acts)


# speedup vs baseline: 1.2292x; 1.0030x over previous
import jax
import jax.numpy as jnp
from jax.experimental import pallas as pl
from jax.experimental.pallas import tpu as pltpu

GROUPS = 8
SUB = 8192
DM = 768
TOPK = 32
ENC_BLK = 2048
NB = SUB // ENC_BLK


def _monotonic_key(v):
    b = jax.lax.bitcast_convert_type(v, jnp.int32)
    flip = jax.lax.shift_right_arithmetic(b, 31) & jnp.int32(0x7FFFFFFF)
    return b ^ flip


def _count_ge(key, t):
    return jnp.sum((key >= t).astype(jnp.int32), axis=1, keepdims=True)


def _descend_bits(key, t, start, n):
    def body(i, t):
        cand = t | (jnp.int32(1) << (start - i))
        return jnp.where(_count_ge(key, cand) >= TOPK, cand, t)
    return jax.lax.fori_loop(0, n, body, t)


def _count_ge_bf16(bkey, cand):
    c = jax.lax.bitcast_convert_type(cand, jnp.float32).astype(jnp.bfloat16)
    a = jnp.where(bkey >= c, jnp.bfloat16(1), jnp.bfloat16(0))
    w = a.shape[1]
    while w > 128:
        a = a[:, : w // 2] + a[:, w // 2:]
        w //= 2
    return jnp.sum(a.astype(jnp.float32), axis=1, keepdims=True)


def _descend_bits_bf16(bkey, t, start, n):
    def body(i, t):
        cand = t | (jnp.int32(1) << (start - i))
        return jnp.where(_count_ge_bf16(bkey, cand) >= TOPK, cand, t)
    return jax.lax.fori_loop(0, n, body, t)


def _select_acts(key, t, p_ref):
    n_ge = _count_ge(key, t)
    col = jax.lax.broadcasted_iota(jnp.int32, key.shape, 1)
    p_ref[...] = jnp.full(p_ref.shape, jnp.int32(8191))

    @pl.when(jnp.any(n_ge > TOPK))
    def _():
        n_gt = jnp.sum((key > t).astype(jnp.int32), axis=1, keepdims=True)
        m = TOPK - n_gt
        tie = key == t

        def body(i, p):
            cand = p | (jnp.int32(1) << (12 - i))
            cnt = jnp.sum((tie & (col < cand)).astype(jnp.int32), axis=1,
                          keepdims=True)
            return jnp.where(cnt < m, cand, p)

        p = jax.lax.fori_loop(0, 13, body,
                              jnp.zeros((key.shape[0], 1), jnp.int32))
        p_ref[...] = jnp.broadcast_to(p, p_ref.shape)

    sel = (key > t) | ((key == t) & (col <= p_ref[:, 0:1]))
    return jnp.where(sel & (key > 0),
                     jax.lax.bitcast_convert_type(key, jnp.float32), 0.0)


def _fused_body(x_ref, w_ref, b_ref, d_ref, out_ref, recon_ref,
                mk_ref, t_ref, p_ref, av_ref, bk_ref):
    g = pl.program_id(0)
    j = pl.program_id(1)

    @pl.when(g < GROUPS)
    def _():
        pre = jax.lax.dot_general(
            x_ref[...], w_ref[0],
            dimension_numbers=(((1,), (1,)), ((), ())),
            preferred_element_type=jnp.float32,
        ) + b_ref[0]
        par = jax.lax.rem(g, 2)
        mk_ref[par, :, pl.ds(j * ENC_BLK, ENC_BLK)] = _monotonic_key(pre)
        bits = jax.lax.bitcast_convert_type(pre, jnp.int32)
        trunc = jax.lax.bitcast_convert_type(
            bits & jnp.int32(-65536), jnp.float32)
        bk_ref[par, :, pl.ds(j * ENC_BLK, ENC_BLK)] = trunc.astype(
            jnp.bfloat16)

    @pl.when((g > 0) & (g <= GROUPS))
    def _():
        par = jax.lax.rem(g - 1, 2)
        key = mk_ref[par]
        bkey = bk_ref[par]
        rows = key.shape[0]

        @pl.when(j == 0)
        def _():
            zero = jnp.zeros((rows, 1), jnp.int32)
            t = jnp.where(_count_ge(key, zero) >= TOPK,
                          zero, jnp.full((rows, 1), jnp.int32(-2147483648)))
            t_ref[...] = jnp.broadcast_to(t, t_ref.shape)

            @pl.when(jnp.all(t >= 0))
            def _():
                tb = _descend_bits_bf16(bkey, t, 30, 7)
                t_ref[...] = jnp.broadcast_to(tb, t_ref.shape)

            @pl.when(jnp.any(t < 0))
            def _():
                tf = _descend_bits(key, t, 30, 7)
                t_ref[...] = jnp.broadcast_to(tf, t_ref.shape)

        @pl.when(j == 1)
        def _():
            t = t_ref[:, 0:1]

            @pl.when(jnp.all(t >= (jnp.int32(1) << 24)))
            def _():
                tb = _descend_bits_bf16(bkey, t, 23, 8)
                t_ref[...] = jnp.broadcast_to(tb, t_ref.shape)

            @pl.when(jnp.any(t < (jnp.int32(1) << 24)))
            def _():
                tf = _descend_bits(key, t, 23, 8)
                t_ref[...] = jnp.broadcast_to(tf, t_ref.shape)

        @pl.when(j == 2)
        def _():
            t = _descend_bits(key, t_ref[:, 0:1], 15, 4)
            t_ref[...] = jnp.broadcast_to(t, t_ref.shape)

        @pl.when(j == 3)
        def _():
            t = t_ref[:, 0:1]
            imin = jnp.int32(-2147483648)
            p_hi = t | jnp.int32(0xFFF)
            n_above = jnp.sum((key > p_hi).astype(jnp.int32), axis=1,
                              keepdims=True)
            r = TOPK - n_above
            act = (key >= t) & (key <= p_hi)
            done = jnp.zeros((rows, 1), jnp.bool_)
            t_fin = jnp.full((rows, 1), imin)
            for _unused in range(3):
                mval = jnp.max(jnp.where(act, key, imin), axis=1,
                               keepdims=True)
                c = jnp.sum((act & (key == mval)).astype(jnp.int32), axis=1,
                            keepdims=True)
                hit = (~done) & (c >= r)
                t_fin = jnp.where(hit, mval, t_fin)
                done = done | hit
                r = jnp.where(done, r, r - c)
                act = act & (key != mval)
            t_ref[...] = jnp.broadcast_to(t_fin, t_ref.shape)

            @pl.when(jnp.any(~done))
            def _():
                t_fb = _descend_bits(key, p_hi ^ jnp.int32(0xFFF), 11, 12)
                t_ref[...] = jnp.broadcast_to(
                    jnp.where(done, t_fin, t_fb), t_ref.shape)

        @pl.when(j == 3)
        def _():
            acts = _select_acts(key, t_ref[:, 0:1], p_ref)
            out_ref[...] = acts
            av_ref[jax.lax.rem(g - 1, 2)] = acts

    @pl.when(g >= 2)
    def _():
        @pl.when((g == 2) & (j == 0))
        def _():
            recon_ref[...] = jnp.zeros_like(recon_ref)

        a_blk = av_ref[jax.lax.rem(g - 2, 2), :, pl.ds(j * ENC_BLK, ENC_BLK)]
        recon_ref[...] += jax.lax.dot_general(
            a_blk, d_ref[0],
            dimension_numbers=(((1,), (1,)), ((), ())),
            preferred_element_type=jnp.float32,
        )


@jax.jit
def kernel(x, enc_W, enc_b, dec_W):
    batch = x.shape[0]
    enc_b3 = enc_b.reshape(GROUPS, 1, SUB)

    full_acts, final_recon = pl.pallas_call(
        _fused_body,
        grid=(GROUPS + 2, NB),
        in_specs=[
            pl.BlockSpec((batch, DM), lambda g, j: (0, 0)),
            pl.BlockSpec((1, ENC_BLK, DM),
                         lambda g, j: (jnp.minimum(g, GROUPS - 1),
                                       jnp.where(g < GROUPS, j, NB - 1), 0)),
            pl.BlockSpec((1, 1, ENC_BLK),
                         lambda g, j: (jnp.minimum(g, GROUPS - 1), 0,
                                       jnp.where(g < GROUPS, j, NB - 1))),
            pl.BlockSpec((1, DM, ENC_BLK),
                         lambda g, j: (jnp.maximum(g - 2, 0), 0,
                                       jnp.where(g >= 2, j, 0))),
        ],
        out_specs=[
            pl.BlockSpec((batch, SUB),
                         lambda g, j: (0, jnp.clip(g - 1, 0, GROUPS - 1))),
            pl.BlockSpec((batch, DM), lambda g, j: (0, 0)),
        ],
        out_shape=[
            jax.ShapeDtypeStruct((batch, GROUPS * SUB), jnp.float32),
            jax.ShapeDtypeStruct((batch, DM), jnp.float32),
        ],
        scratch_shapes=[
            pltpu.VMEM((2, batch, SUB), jnp.int32),
            pltpu.VMEM((batch, 128), jnp.int32),
            pltpu.VMEM((batch, 128), jnp.int32),
            pltpu.VMEM((2, batch, SUB), jnp.float32),
            pltpu.VMEM((2, batch, SUB), jnp.bfloat16),
        ],
    )(x, enc_W, enc_b3, dec_W)

    return (final_recon, full_acts)


# speedup vs baseline: 1.2435x; 1.0116x over previous
import jax
import jax.numpy as jnp
from jax.experimental import pallas as pl
from jax.experimental.pallas import tpu as pltpu

GROUPS = 8
SUB = 8192
DM = 768
TOPK = 32
ENC_BLK = 2048
NB = SUB // ENC_BLK


def _monotonic_key(v):
    b = jax.lax.bitcast_convert_type(v, jnp.int32)
    flip = jax.lax.shift_right_arithmetic(b, 31) & jnp.int32(0x7FFFFFFF)
    return b ^ flip


def _count_ge(key, t):
    return jnp.sum((key >= t).astype(jnp.int32), axis=1, keepdims=True)


def _descend_bits(key, t, start, n):
    def body(i, t):
        cand = t | (jnp.int32(1) << (start - i))
        return jnp.where(_count_ge(key, cand) >= TOPK, cand, t)
    return jax.lax.fori_loop(0, n, body, t)


def _count_ge_bf16(bkey, cand):
    c = jax.lax.bitcast_convert_type(cand, jnp.float32).astype(jnp.bfloat16)
    a = jnp.where(bkey >= c, jnp.bfloat16(1), jnp.bfloat16(0))
    w = a.shape[1]
    while w > 128:
        a = a[:, : w // 2] + a[:, w // 2:]
        w //= 2
    return jnp.sum(a.astype(jnp.float32), axis=1, keepdims=True)


def _descend_bits_bf16(bkey, t, start, n):
    def body(i, t):
        cand = t | (jnp.int32(1) << (start - i))
        return jnp.where(_count_ge_bf16(bkey, cand) >= TOPK, cand, t)
    return jax.lax.fori_loop(0, n, body, t)


def _select_acts(key, t, p_ref, tie_any):
    col = jax.lax.broadcasted_iota(jnp.int32, key.shape, 1)
    p_ref[...] = jnp.full(p_ref.shape, jnp.int32(8191))

    @pl.when(tie_any)
    def _():
        n_gt = jnp.sum((key > t).astype(jnp.int32), axis=1, keepdims=True)
        m = TOPK - n_gt
        tie = key == t

        def body(i, p):
            cand = p | (jnp.int32(1) << (12 - i))
            cnt = jnp.sum((tie & (col < cand)).astype(jnp.int32), axis=1,
                          keepdims=True)
            return jnp.where(cnt < m, cand, p)

        p = jax.lax.fori_loop(0, 13, body,
                              jnp.zeros((key.shape[0], 1), jnp.int32))
        p_ref[...] = jnp.broadcast_to(p, p_ref.shape)

    sel = (key > t) | ((key == t) & (col <= p_ref[:, 0:1]))
    return jnp.where(sel & (key > 0),
                     jax.lax.bitcast_convert_type(key, jnp.float32), 0.0)


def _fused_body(x_ref, w_ref, b_ref, d_ref, out_ref, recon_ref,
                mk_ref, t_ref, p_ref, av_ref, bk_ref):
    g = pl.program_id(0)
    j = pl.program_id(1)

    @pl.when(g < GROUPS)
    def _():
        pre = jax.lax.dot_general(
            x_ref[...], w_ref[0],
            dimension_numbers=(((1,), (1,)), ((), ())),
            preferred_element_type=jnp.float32,
        ) + b_ref[0]
        par = jax.lax.rem(g, 2)
        mk_ref[par, :, pl.ds(j * ENC_BLK, ENC_BLK)] = _monotonic_key(pre)
        bits = jax.lax.bitcast_convert_type(pre, jnp.int32)
        trunc = jax.lax.bitcast_convert_type(
            bits & jnp.int32(-65536), jnp.float32)
        bk_ref[par, :, pl.ds(j * ENC_BLK, ENC_BLK)] = trunc.astype(
            jnp.bfloat16)

    @pl.when((g > 0) & (g <= GROUPS))
    def _():
        par = jax.lax.rem(g - 1, 2)
        key = mk_ref[par]
        bkey = bk_ref[par]
        rows = key.shape[0]

        @pl.when(j == 0)
        def _():
            zero = jnp.zeros((rows, 1), jnp.int32)
            t = jnp.where(_count_ge(key, zero) >= TOPK,
                          zero, jnp.full((rows, 1), jnp.int32(-2147483648)))
            t_ref[...] = jnp.broadcast_to(t, t_ref.shape)

            @pl.when(jnp.all(t >= 0))
            def _():
                tb = _descend_bits_bf16(bkey, t, 30, 7)
                t_ref[...] = jnp.broadcast_to(tb, t_ref.shape)

            @pl.when(jnp.any(t < 0))
            def _():
                tf = _descend_bits(key, t, 30, 7)
                t_ref[...] = jnp.broadcast_to(tf, t_ref.shape)

        @pl.when(j == 1)
        def _():
            t = t_ref[:, 0:1]

            @pl.when(jnp.all(t >= (jnp.int32(1) << 24)))
            def _():
                tb = _descend_bits_bf16(bkey, t, 23, 8)
                t_ref[...] = jnp.broadcast_to(tb, t_ref.shape)

            @pl.when(jnp.any(t < (jnp.int32(1) << 24)))
            def _():
                tf = _descend_bits(key, t, 23, 8)
                t_ref[...] = jnp.broadcast_to(tf, t_ref.shape)

        @pl.when(j == 2)
        def _():
            t = _descend_bits(key, t_ref[:, 0:1], 15, 4)
            t_ref[...] = jnp.broadcast_to(t, t_ref.shape)

        @pl.when(j == 3)
        def _():
            t = t_ref[:, 0:1]
            imin = jnp.int32(-2147483648)
            p_hi = t | jnp.int32(0xFFF)
            n_above = jnp.sum((key > p_hi).astype(jnp.int32), axis=1,
                              keepdims=True)
            r = TOPK - n_above
            act = (key >= t) & (key <= p_hi)
            done = jnp.zeros((rows, 1), jnp.bool_)
            extras = jnp.zeros((rows, 1), jnp.bool_)
            t_fin = jnp.full((rows, 1), imin)
            for _unused in range(3):
                mval = jnp.max(jnp.where(act, key, imin), axis=1,
                               keepdims=True)
                c = jnp.sum((act & (key == mval)).astype(jnp.int32), axis=1,
                            keepdims=True)
                hit = (~done) & (c >= r)
                extras = extras | (hit & (c > r))
                t_fin = jnp.where(hit, mval, t_fin)
                done = done | hit
                r = jnp.where(done, r, r - c)
                act = act & (key != mval)
            t_ref[...] = jnp.broadcast_to(t_fin, t_ref.shape)
            some_undone = jnp.any(~done)

            @pl.when(some_undone)
            def _():
                t_fb = _descend_bits(key, p_hi ^ jnp.int32(0xFFF), 11, 12)
                t_ref[...] = jnp.broadcast_to(
                    jnp.where(done, t_fin, t_fb), t_ref.shape)

            tie_any = jnp.any(extras) | some_undone
            acts = _select_acts(key, t_ref[:, 0:1], p_ref, tie_any)
            out_ref[...] = acts
            av_ref[jax.lax.rem(g - 1, 2)] = acts

    @pl.when(g >= 2)
    def _():
        @pl.when((g == 2) & (j == 0))
        def _():
            recon_ref[...] = jnp.zeros_like(recon_ref)

        a_blk = av_ref[jax.lax.rem(g - 2, 2), :, pl.ds(j * ENC_BLK, ENC_BLK)]
        recon_ref[...] += jax.lax.dot_general(
            a_blk, d_ref[0],
            dimension_numbers=(((1,), (1,)), ((), ())),
            preferred_element_type=jnp.float32,
        )


@jax.jit
def kernel(x, enc_W, enc_b, dec_W):
    batch = x.shape[0]
    enc_b3 = enc_b.reshape(GROUPS, 1, SUB)

    full_acts, final_recon = pl.pallas_call(
        _fused_body,
        grid=(GROUPS + 2, NB),
        in_specs=[
            pl.BlockSpec((batch, DM), lambda g, j: (0, 0)),
            pl.BlockSpec((1, ENC_BLK, DM),
                         lambda g, j: (jnp.minimum(g, GROUPS - 1),
                                       jnp.where(g < GROUPS, j, NB - 1), 0)),
            pl.BlockSpec((1, 1, ENC_BLK),
                         lambda g, j: (jnp.minimum(g, GROUPS - 1), 0,
                                       jnp.where(g < GROUPS, j, NB - 1))),
            pl.BlockSpec((1, DM, ENC_BLK),
                         lambda g, j: (jnp.maximum(g - 2, 0), 0,
                                       jnp.where(g >= 2, j, 0))),
        ],
        out_specs=[
            pl.BlockSpec((batch, SUB),
                         lambda g, j: (0, jnp.clip(g - 1, 0, GROUPS - 1))),
            pl.BlockSpec((batch, DM), lambda g, j: (0, 0)),
        ],
        out_shape=[
            jax.ShapeDtypeStruct((batch, GROUPS * SUB), jnp.float32),
            jax.ShapeDtypeStruct((batch, DM), jnp.float32),
        ],
        scratch_shapes=[
            pltpu.VMEM((2, batch, SUB), jnp.int32),
            pltpu.VMEM((batch, 128), jnp.int32),
            pltpu.VMEM((batch, 128), jnp.int32),
            pltpu.VMEM((2, batch, SUB), jnp.float32),
            pltpu.VMEM((2, batch, SUB), jnp.bfloat16),
        ],
    )(x, enc_W, enc_b3, dec_W)

    return (final_recon, full_acts)
